# baseline (speedup 1.0000x reference)
import functools

import jax
import jax.numpy as jnp
from jax import lax
from jax.experimental import pallas as pl
from jax.experimental.pallas import tpu as pltpu
from jax.experimental.pallas import tpu_sc as plsc

_VOCAB = 500
_EMB_DIM = 10
_N_CLASSES = 2
_BATCH = 16384
_HIST = 200
_CPAD = 512

_NC = 2
_NS = 16
_NW = _NC * _NS
_L = 16

_ROWS_PER_W = _BATCH // _NW
_CHUNK_ROWS = 128
_N_CHUNKS = _ROWS_PER_W // _CHUNK_ROWS
_VECS_PER_ROW = _HIST // _L
_TAIL = _HIST - _VECS_PER_ROW * _L


def _coef_body(emb_ref, lin_ref, bias_ref, out_ref):
    emb = emb_ref[...]
    nsq = jnp.maximum(jnp.sum(emb * emb, axis=1, keepdims=True), 1e-24)
    y = lax.rsqrt(nsq)
    y = y * (1.5 - 0.5 * nsq * y * y)
    y = y * (1.5 - 0.5 * nsq * y * y)
    scale = jnp.minimum(1.0, y)
    table = emb * scale
    w = lin_ref[...].astype(jnp.bfloat16).astype(jnp.float32)
    coef = lax.dot_general(
        w, table, (((1,), (1,)), ((), ())),
        preferred_element_type=jnp.float32,
        precision=lax.Precision.HIGHEST)
    out_ref[...] = coef + bias_ref[...] * (1.0 / _HIST)


def _compute_coef(emb_weight, lin_weight, lin_bias):
    return pl.pallas_call(
        _coef_body,
        out_shape=jax.ShapeDtypeStruct((_N_CLASSES, _VOCAB), jnp.float32),
    )(emb_weight, lin_weight, lin_bias.reshape(_N_CLASSES, 1))


def _sc_body(in_hbm, coef_hbm, lab_hbm, out_hbm,
             coef_v, lab_v, hist_v, buf0, buf1, out_v, sem0, sem1):
    wid = lax.axis_index("s") * _NC + lax.axis_index("c")
    row_base = wid * _ROWS_PER_W

    pltpu.sync_copy(coef_hbm, coef_v)
    pltpu.sync_copy(lab_hbm.at[pl.ds(row_base, _ROWS_PER_W)], lab_v)

    lane = lax.iota(jnp.int32, _L)
    tail_mask = lane >= (_L - _TAIL)

    def scale_lab(i, carry):
        lab_v[pl.ds(i * _L, _L)] = lab_v[pl.ds(i * _L, _L)] * _CPAD
        return carry
    lax.fori_loop(0, _ROWS_PER_W // _L, scale_lab, 0)

    ones = jnp.ones((_L,), jnp.float32)

    def zero_hist(i, carry):
        hist_v[pl.ds(i * _L, _L)] = jnp.zeros((_L,), jnp.float32)
        return carry
    lax.fori_loop(0, (_N_CLASSES * _CPAD) // _L, zero_hist, 0)

    bufs = (buf0, buf1)
    sems = (sem0, sem1)

    def chunk_src(chunk):
        return in_hbm.at[pl.ds(row_base + chunk * _CHUNK_ROWS, _CHUNK_ROWS)]

    copies = [None, None]
    copies[0] = pltpu.async_copy(chunk_src(0), bufs[0], sems[0])

    acc = jnp.zeros((_L,), jnp.float32)
    for c in range(_N_CHUNKS):
        b = c & 1
        if c + 1 < _N_CHUNKS:
            nb = (c + 1) & 1
            copies[nb] = pltpu.async_copy(chunk_src(c + 1), bufs[nb], sems[nb])
        copies[b].wait()
        buf = bufs[b]
        chunk_row0 = c * _CHUNK_ROWS

        def row_step(r, carry, buf=buf, chunk_row0=chunk_row0):
            off = plsc.load_gather(
                lab_v, [jnp.full((_L,), chunk_row0, jnp.int32) + r])
            for j in range(_VECS_PER_ROW):
                iv = buf[r, pl.ds(j * _L, _L)]
                plsc.addupdate_scatter(hist_v, [iv + off], ones)
            ivt = buf[r, pl.ds(_HIST - _L, _L)]
            plsc.addupdate_scatter(hist_v, [ivt + off], ones, mask=tail_mask)
            return carry

        lax.fori_loop(0, _CHUNK_ROWS, row_step, 0, unroll=2)

    def dot_step(i, a):
        return a + hist_v[pl.ds(i * _L, _L)] * coef_v[pl.ds(i * _L, _L)]
    acc = lax.fori_loop(0, (_N_CLASSES * _CPAD) // _L, dot_step, acc,
                        unroll=4)

    out_v[...] = acc
    pltpu.sync_copy(out_v, out_hbm.at[wid])


@functools.partial(
    pl.kernel,
    out_type=jax.ShapeDtypeStruct((_NW, _L), jnp.float32),
    mesh=plsc.VectorSubcoreMesh(core_axis_name="c", subcore_axis_name="s"),
    compiler_params=pltpu.CompilerParams(
        needs_layout_passes=False, use_tc_tiling_on_sc=True),
    scratch_types=[
        pltpu.VMEM((_N_CLASSES * _CPAD,), jnp.float32),
        pltpu.VMEM((_ROWS_PER_W,), jnp.int32),
        pltpu.VMEM((_N_CLASSES * _CPAD,), jnp.float32),
        pltpu.VMEM((_CHUNK_ROWS, _HIST), jnp.int32),
        pltpu.VMEM((_CHUNK_ROWS, _HIST), jnp.int32),
        pltpu.VMEM((_L,), jnp.float32),
        pltpu.SemaphoreType.DMA,
        pltpu.SemaphoreType.DMA,
    ],
)
def _sc_gather_sum(in_hbm, coef_hbm, lab_hbm, out_hbm,
                   coef_v, lab_v, hist_v, buf0, buf1, out_v, sem0, sem1):
    _sc_body(in_hbm, coef_hbm, lab_hbm, out_hbm,
             coef_v, lab_v, hist_v, buf0, buf1, out_v, sem0, sem1)


def kernel(input, labels, emb_weight, lin_weight, lin_bias):
    coef = _compute_coef(emb_weight, lin_weight, lin_bias)
    coef_flat = jnp.pad(coef, ((0, 0), (0, _CPAD - _VOCAB))).reshape(-1)
    partials = _sc_gather_sum(input, coef_flat, labels)
    return -jnp.sum(partials) / _BATCH


# speedup vs baseline: 1.7316x; 1.7316x over previous
import functools

import jax
import jax.numpy as jnp
from jax import lax
from jax.experimental import pallas as pl
from jax.experimental.pallas import tpu as pltpu
from jax.experimental.pallas import tpu_sc as plsc

_VOCAB = 500
_EMB_DIM = 10
_N_CLASSES = 2
_BATCH = 16384
_HIST = 200
_CPAD = 512

_NC = 2
_NS = 16
_NW = _NC * _NS
_L = 16

_ROWS_PER_W = _BATCH // _NW
_CHUNK_ROWS = 64
_N_CHUNKS = _ROWS_PER_W // _CHUNK_ROWS
_VECS_PER_ROW = _HIST // _L
_TAIL = _HIST - _VECS_PER_ROW * _L


def _coef_body(emb_ref, lin_ref, bias_ref, out_ref):
    emb = emb_ref[...]
    nsq = jnp.maximum(jnp.sum(emb * emb, axis=1, keepdims=True), 1e-24)
    y = lax.rsqrt(nsq)
    y = y * (1.5 - 0.5 * nsq * y * y)
    y = y * (1.5 - 0.5 * nsq * y * y)
    scale = jnp.minimum(1.0, y)
    table = emb * scale
    w = lin_ref[...].astype(jnp.bfloat16).astype(jnp.float32)
    coef = lax.dot_general(
        w, table, (((1,), (1,)), ((), ())),
        preferred_element_type=jnp.float32,
        precision=lax.Precision.HIGHEST)
    out_ref[...] = coef + bias_ref[...] * (1.0 / _HIST)


def _compute_coef(emb_weight, lin_weight, lin_bias):
    return pl.pallas_call(
        _coef_body,
        out_shape=jax.ShapeDtypeStruct((_N_CLASSES, _VOCAB), jnp.float32),
    )(emb_weight, lin_weight, lin_bias.reshape(_N_CLASSES, 1))


def _sc_body(in_hbm, coef_hbm, lab_hbm, out_hbm,
             coef_v, lab_v, buf0, buf1, out_v, sem0, sem1):
    wid = lax.axis_index("s") * _NC + lax.axis_index("c")
    row_base = wid * _ROWS_PER_W

    bufs = (buf0, buf1)
    sems = (sem0, sem1)

    def chunk_src(chunk):
        return in_hbm.at[pl.ds(row_base + chunk * _CHUNK_ROWS, _CHUNK_ROWS)]

    copies = [None, None]
    copies[0] = pltpu.async_copy(chunk_src(0), bufs[0], sems[0])

    pltpu.sync_copy(coef_hbm, coef_v)
    pltpu.sync_copy(lab_hbm.at[pl.ds(row_base, _ROWS_PER_W)], lab_v)

    lane = lax.iota(jnp.int32, _L)
    tail_mask = lane >= (_L - _TAIL)

    def scale_lab(i, carry):
        lab_v[pl.ds(i * _L, _L)] = lab_v[pl.ds(i * _L, _L)] * _CPAD
        return carry
    lax.fori_loop(0, _ROWS_PER_W // _L, scale_lab, 0)

    acc = jnp.zeros((_L,), jnp.float32)
    for c in range(_N_CHUNKS):
        b = c & 1
        if c + 1 < _N_CHUNKS:
            nb = (c + 1) & 1
            copies[nb] = pltpu.async_copy(chunk_src(c + 1), bufs[nb], sems[nb])
        copies[b].wait()
        buf = bufs[b]
        chunk_row0 = c * _CHUNK_ROWS

        def row_step(r, a, buf=buf, chunk_row0=chunk_row0):
            off = plsc.load_gather(
                lab_v, [jnp.full((_L,), chunk_row0, jnp.int32) + r])
            gs = []
            for j in range(_VECS_PER_ROW):
                iv = buf[r, pl.ds(j * _L, _L)]
                gs.append(plsc.load_gather(coef_v, [iv + off]))
            ivt = buf[r, pl.ds(_HIST - _L, _L)]
            gt = plsc.load_gather(coef_v, [ivt + off])
            gs.append(jnp.where(tail_mask, gt, 0.0))
            while len(gs) > 1:
                rest = [gs[-1]] if len(gs) % 2 else []
                gs = [x + y for x, y in zip(gs[::2], gs[1::2])] + rest
            return a + gs[0]

        acc = lax.fori_loop(0, _CHUNK_ROWS, row_step, acc, unroll=4)

    out_v[...] = acc
    pltpu.sync_copy(out_v, out_hbm.at[wid])


@functools.partial(
    pl.kernel,
    out_type=jax.ShapeDtypeStruct((_NW, _L), jnp.float32),
    mesh=plsc.VectorSubcoreMesh(core_axis_name="c", subcore_axis_name="s"),
    compiler_params=pltpu.CompilerParams(
        needs_layout_passes=False, use_tc_tiling_on_sc=True),
    scratch_types=[
        pltpu.VMEM((_N_CLASSES * _CPAD,), jnp.float32),
        pltpu.VMEM((_ROWS_PER_W,), jnp.int32),
        pltpu.VMEM((_CHUNK_ROWS, _HIST), jnp.int32),
        pltpu.VMEM((_CHUNK_ROWS, _HIST), jnp.int32),
        pltpu.VMEM((_L,), jnp.float32),
        pltpu.SemaphoreType.DMA,
        pltpu.SemaphoreType.DMA,
    ],
)
def _sc_gather_sum(in_hbm, coef_hbm, lab_hbm, out_hbm,
                   coef_v, lab_v, buf0, buf1, out_v, sem0, sem1):
    _sc_body(in_hbm, coef_hbm, lab_hbm, out_hbm,
             coef_v, lab_v, buf0, buf1, out_v, sem0, sem1)


def kernel(input, labels, emb_weight, lin_weight, lin_bias):
    coef = _compute_coef(emb_weight, lin_weight, lin_bias)
    coef_flat = jnp.pad(coef, ((0, 0), (0, _CPAD - _VOCAB))).reshape(-1)
    partials = _sc_gather_sum(input, coef_flat, labels)
    return -jnp.sum(partials) / _BATCH
